# baseline (speedup 1.0000x reference)
import functools

import jax
import jax.numpy as jnp
from jax import lax
from jax.experimental import pallas as pl
from jax.experimental.pallas import tpu as pltpu
from jax.experimental.pallas import tpu_sc as plsc

_N = 8192
_K = 8192
_C = 256
_RB = 256
_KB = 1024
_RSTEPS = _N // _RB
_KSTEPS = _K // _KB

_NW = 32
_BPW = _N // _NW
_JCH = _BPW // 128


def _vq_body(x_ref, cb_ref, idx_ref, loss_ref,
             c2_ref, chm_ref, chi_ref, accv_ref, acci_ref, acc_ref):
    i = pl.program_id(0)
    j = pl.program_id(1)
    x = x_ref[...]
    cb = cb_ref[...]

    @pl.when(i == 0)
    def _():
        ones = jnp.ones((1, _C), jnp.float32)
        c2 = lax.dot_general(ones, cb * cb, (((1,), (1,)), ((), ())),
                             preferred_element_type=jnp.float32,
                             precision=lax.Precision.HIGHEST)
        c2_ref[:, pl.ds(j * _KB, _KB)] = c2

    xc = lax.dot_general(x.astype(jnp.bfloat16), cb.astype(jnp.bfloat16),
                         (((1,), (1,)), ((), ())),
                         preferred_element_type=jnp.float32)
    c2 = c2_ref[:, pl.ds(j * _KB, _KB)]
    x2 = jnp.sum(x * x, axis=1, keepdims=True)
    dist = (x2 - 2.0 * xc) + c2
    vmin = jnp.min(dist, axis=1, keepdims=True)
    cols = lax.broadcasted_iota(jnp.int32, (_RB, _KB), 1) + j * _KB
    pidx = jnp.min(jnp.where(dist == vmin, cols, _K),
                   axis=1, keepdims=True)

    @pl.when(j % 2 == 0)
    def _():
        chm_ref[...] = vmin
        chi_ref[...] = pidx

    @pl.when(j % 2 == 1)
    def _():
        prev = chm_ref[...]
        better = vmin < prev
        chm = jnp.where(better, vmin, prev)
        chi = jnp.where(better, pidx, chi_ref[...])

        @pl.when(j == 1)
        def _():
            accv_ref[...] = chm
            acci_ref[...] = chi

        @pl.when(j > 1)
        def _():
            thr = accv_ref[...].astype(jnp.bfloat16).astype(jnp.float32)
            take = chm < thr
            accv_ref[...] = jnp.where(take, chm, accv_ref[...])
            acci_ref[...] = jnp.where(take, chi, acci_ref[...])

    @pl.when(jnp.logical_and(i == 0, j == 0))
    def _():
        acc_ref[...] = jnp.zeros((1, 1), jnp.float32)

    @pl.when(j == _KSTEPS - 1)
    def _():
        idx_ref[...] = acci_ref[...]
        acc_ref[...] = acc_ref[...] + jnp.sum(accv_ref[...])

    @pl.when(jnp.logical_and(i == _RSTEPS - 1, j == _KSTEPS - 1))
    def _():
        loss_ref[...] = acc_ref[...] * (1.0 / (_N * _C))


def _vq_tc(x_flat, codebook):
    return pl.pallas_call(
        _vq_body,
        grid=(_RSTEPS, _KSTEPS),
        in_specs=[
            pl.BlockSpec((_RB, _C), lambda i, j: (i, 0)),
            pl.BlockSpec((_KB, _C), lambda i, j: (j, 0)),
        ],
        out_specs=[
            pl.BlockSpec((_RB, 1), lambda i, j: (i, 0)),
            pl.BlockSpec((1, 1), lambda i, j: (0, 0)),
        ],
        out_shape=[
            jax.ShapeDtypeStruct((_N, 1), jnp.int32),
            jax.ShapeDtypeStruct((1, 1), jnp.float32),
        ],
        scratch_shapes=[
            pltpu.VMEM((1, _K), jnp.float32),
            pltpu.VMEM((_RB, 1), jnp.float32),
            pltpu.VMEM((_RB, 1), jnp.int32),
            pltpu.VMEM((_RB, 1), jnp.float32),
            pltpu.VMEM((_RB, 1), jnp.int32),
            pltpu.VMEM((1, 1), jnp.float32),
        ],
        compiler_params=pltpu.CompilerParams(
            dimension_semantics=("arbitrary", "arbitrary"),
        ),
    )(x_flat, codebook)


def _sc_gather(codebook, idx3):
    mesh = plsc.VectorSubcoreMesh(core_axis_name="c", subcore_axis_name="s")

    @functools.partial(
        pl.kernel,
        out_type=jax.ShapeDtypeStruct((_N, _C), jnp.float32),
        mesh=mesh,
        scratch_types=[
            pltpu.VMEM((_JCH, 128), jnp.int32),
            pltpu.VMEM((128, _C), jnp.float32),
            pltpu.SemaphoreType.DMA,
        ],
    )
    def gk(cb_hbm, idx_hbm, out_hbm, idx_v, rows_v, sem):
        wid = lax.axis_index("s") * 2 + lax.axis_index("c")
        base = wid * _BPW
        pltpu.sync_copy(idx_hbm.at[wid], idx_v)
        for j in range(_JCH):
            pltpu.async_copy(cb_hbm.at[idx_v.at[j]], rows_v, sem).wait()
            pltpu.sync_copy(rows_v, out_hbm.at[pl.ds(base + j * 128, 128)])

    return gk(codebook, idx3)


def kernel(x, codebook):
    B, T, C = x.shape
    x_flat = x.reshape(-1, C)
    idx2, loss = _vq_tc(x_flat, codebook)
    idx_flat = idx2[:, 0]
    qe_flat = _sc_gather(codebook, idx_flat.reshape(_NW, _JCH, 128))
    qe = qe_flat.reshape(B, T, C)
    return (qe, loss[0, 0], idx_flat.reshape(B, T))


# speedup vs baseline: 1.3917x; 1.3917x over previous
import functools

import jax
import jax.numpy as jnp
from jax import lax
from jax.experimental import pallas as pl
from jax.experimental.pallas import tpu as pltpu
from jax.experimental.pallas import tpu_sc as plsc

_N = 8192
_K = 8192
_C = 256
_RB = 256
_KB = 2048
_RSTEPS = _N // _RB
_KSTEPS = _K // _KB

_NW = 32
_BPW = _N // _NW
_JCH = _BPW // 128


def _vq_body(x_ref, cb_ref, idx_ref, loss_ref,
             c2_ref, accv_ref, acci_ref, acc_ref):
    i = pl.program_id(0)
    j = pl.program_id(1)
    x = x_ref[...]
    cb = cb_ref[...]

    @pl.when(i == 0)
    def _():
        ones = jnp.ones((1, _C), jnp.float32)
        c2 = lax.dot_general(ones, cb * cb, (((1,), (1,)), ((), ())),
                             preferred_element_type=jnp.float32,
                             precision=lax.Precision.HIGHEST)
        c2_ref[:, pl.ds(j * _KB, _KB)] = c2

    xc = lax.dot_general(x.astype(jnp.bfloat16), cb.astype(jnp.bfloat16),
                         (((1,), (1,)), ((), ())),
                         preferred_element_type=jnp.float32)
    c2 = c2_ref[:, pl.ds(j * _KB, _KB)]
    x2 = jnp.sum(x * x, axis=1, keepdims=True)
    dist = (x2 - 2.0 * xc) + c2
    vmin = jnp.min(dist, axis=1, keepdims=True)
    cols = lax.broadcasted_iota(jnp.int32, (_RB, _KB), 1) + j * _KB
    pidx = jnp.min(jnp.where(dist == vmin, cols, _K),
                   axis=1, keepdims=True)

    @pl.when(j == 0)
    def _():
        accv_ref[...] = vmin
        acci_ref[...] = pidx

    @pl.when(j > 0)
    def _():
        thr = accv_ref[...].astype(jnp.bfloat16).astype(jnp.float32)
        take = vmin < thr
        accv_ref[...] = jnp.where(take, vmin, accv_ref[...])
        acci_ref[...] = jnp.where(take, pidx, acci_ref[...])

    @pl.when(jnp.logical_and(i == 0, j == 0))
    def _():
        acc_ref[...] = jnp.zeros((1, 1), jnp.float32)

    @pl.when(j == _KSTEPS - 1)
    def _():
        idx_ref[...] = acci_ref[...]
        acc_ref[...] = acc_ref[...] + jnp.sum(accv_ref[...])

    @pl.when(jnp.logical_and(i == _RSTEPS - 1, j == _KSTEPS - 1))
    def _():
        loss_ref[...] = acc_ref[...] * (1.0 / (_N * _C))


def _vq_tc(x_flat, codebook):
    return pl.pallas_call(
        _vq_body,
        grid=(_RSTEPS, _KSTEPS),
        in_specs=[
            pl.BlockSpec((_RB, _C), lambda i, j: (i, 0)),
            pl.BlockSpec((_KB, _C), lambda i, j: (j, 0)),
        ],
        out_specs=[
            pl.BlockSpec((_RB, 1), lambda i, j: (i, 0)),
            pl.BlockSpec((1, 1), lambda i, j: (0, 0)),
        ],
        out_shape=[
            jax.ShapeDtypeStruct((_N, 1), jnp.int32),
            jax.ShapeDtypeStruct((1, 1), jnp.float32),
        ],
        scratch_shapes=[
            pltpu.VMEM((1, _K), jnp.float32),
            pltpu.VMEM((_RB, 1), jnp.float32),
            pltpu.VMEM((_RB, 1), jnp.int32),
            pltpu.VMEM((1, 1), jnp.float32),
        ],
        compiler_params=pltpu.CompilerParams(
            dimension_semantics=("arbitrary", "arbitrary"),
        ),
    )(x_flat, codebook)


def _sc_gather(codebook, idx3):
    mesh = plsc.VectorSubcoreMesh(core_axis_name="c", subcore_axis_name="s")

    @functools.partial(
        pl.kernel,
        out_type=jax.ShapeDtypeStruct((_N, _C), jnp.float32),
        mesh=mesh,
        scratch_types=[
            pltpu.VMEM((_JCH, 128), jnp.int32),
            pltpu.VMEM((128, _C), jnp.float32),
            pltpu.SemaphoreType.DMA,
        ],
    )
    def gk(cb_hbm, idx_hbm, out_hbm, idx_v, rows_v, sem):
        wid = lax.axis_index("s") * 2 + lax.axis_index("c")
        base = wid * _BPW
        pltpu.sync_copy(idx_hbm.at[wid], idx_v)
        for j in range(_JCH):
            pltpu.async_copy(cb_hbm.at[idx_v.at[j]], rows_v, sem).wait()
            pltpu.sync_copy(rows_v, out_hbm.at[pl.ds(base + j * 128, 128)])

    return gk(codebook, idx3)


def kernel(x, codebook):
    B, T, C = x.shape
    x_flat = x.reshape(-1, C)
    idx2, loss = _vq_tc(x_flat, codebook)
    idx_flat = idx2[:, 0]
    qe_flat = _sc_gather(codebook, idx_flat.reshape(_NW, _JCH, 128))
    qe = qe_flat.reshape(B, T, C)
    return (qe, loss[0, 0], idx_flat.reshape(B, T))


# speedup vs baseline: 1.4410x; 1.0354x over previous
import functools

import jax
import jax.numpy as jnp
from jax import lax
from jax.experimental import pallas as pl
from jax.experimental.pallas import tpu as pltpu
from jax.experimental.pallas import tpu_sc as plsc

_N = 8192
_K = 8192
_C = 256
_RB = 256
_KB = 2048
_RSTEPS = _N // _RB
_KSTEPS = _K // _KB

_NW = 32
_BPW = _N // _NW
_JCH = _BPW // 128


def _vq_body(x_ref, cb_ref, idx_ref, loss_ref,
             c2_ref, colf_ref, x2_ref, dist_ref, accv_ref, acci_ref, acc_ref):
    i = pl.program_id(0)
    j = pl.program_id(1)
    x = x_ref[...]
    cb = cb_ref[...]

    @pl.when(i == 0)
    def _():
        ones = jnp.ones((1, _C), jnp.float32)
        c2 = lax.dot_general(ones, cb * cb, (((1,), (1,)), ((), ())),
                             preferred_element_type=jnp.float32,
                             precision=lax.Precision.HIGHEST)
        c2_ref[:, pl.ds(j * _KB, _KB)] = c2

    @pl.when(jnp.logical_and(i == 0, j == 0))
    def _():
        colf_ref[...] = lax.broadcasted_iota(
            jnp.int32, (1, _KB), 1).astype(jnp.float32)

    @pl.when(j == 0)
    def _():
        x2_ref[...] = jnp.sum(x * x, axis=1, keepdims=True)

    xc = lax.dot_general(x.astype(jnp.bfloat16), cb.astype(jnp.bfloat16),
                         (((1,), (1,)), ((), ())),
                         preferred_element_type=jnp.float32)
    c2 = c2_ref[:, pl.ds(j * _KB, _KB)]
    dist_ref[...] = (x2_ref[...] - 2.0 * xc) + c2
    dist = dist_ref[...]
    vmin = jnp.min(dist, axis=1, keepdims=True)
    idxf = jnp.min(jnp.where(dist == vmin, colf_ref[...], float(_K)),
                   axis=1, keepdims=True)
    pidx = idxf.astype(jnp.int32) + j * _KB

    @pl.when(j == 0)
    def _():
        accv_ref[...] = vmin
        acci_ref[...] = pidx

    @pl.when(j > 0)
    def _():
        thr = accv_ref[...].astype(jnp.bfloat16).astype(jnp.float32)
        take = vmin < thr
        accv_ref[...] = jnp.where(take, vmin, accv_ref[...])
        acci_ref[...] = jnp.where(take, pidx, acci_ref[...])

    @pl.when(jnp.logical_and(i == 0, j == 0))
    def _():
        acc_ref[...] = jnp.zeros((1, 1), jnp.float32)

    @pl.when(j == _KSTEPS - 1)
    def _():
        idx_ref[...] = acci_ref[...]
        acc_ref[...] = acc_ref[...] + jnp.sum(accv_ref[...])

    @pl.when(jnp.logical_and(i == _RSTEPS - 1, j == _KSTEPS - 1))
    def _():
        loss_ref[...] = acc_ref[...] * (1.0 / (_N * _C))


def _vq_tc(x_flat, codebook):
    return pl.pallas_call(
        _vq_body,
        grid=(_RSTEPS, _KSTEPS),
        in_specs=[
            pl.BlockSpec((_RB, _C), lambda i, j: (i, 0)),
            pl.BlockSpec((_KB, _C), lambda i, j: (j, 0)),
        ],
        out_specs=[
            pl.BlockSpec((_RB, 1), lambda i, j: (i, 0)),
            pl.BlockSpec((1, 1), lambda i, j: (0, 0)),
        ],
        out_shape=[
            jax.ShapeDtypeStruct((_N, 1), jnp.int32),
            jax.ShapeDtypeStruct((1, 1), jnp.float32),
        ],
        scratch_shapes=[
            pltpu.VMEM((1, _K), jnp.float32),
            pltpu.VMEM((1, _KB), jnp.float32),
            pltpu.VMEM((_RB, 1), jnp.float32),
            pltpu.VMEM((_RB, _KB), jnp.float32),
            pltpu.VMEM((_RB, 1), jnp.float32),
            pltpu.VMEM((_RB, 1), jnp.int32),
            pltpu.VMEM((1, 1), jnp.float32),
        ],
        compiler_params=pltpu.CompilerParams(
            dimension_semantics=("arbitrary", "arbitrary"),
        ),
    )(x_flat, codebook)


def _sc_gather(codebook, idx3):
    mesh = plsc.VectorSubcoreMesh(core_axis_name="c", subcore_axis_name="s")

    @functools.partial(
        pl.kernel,
        out_type=jax.ShapeDtypeStruct((_N, _C), jnp.float32),
        mesh=mesh,
        scratch_types=[
            pltpu.VMEM((_JCH, 128), jnp.int32),
            pltpu.VMEM((128, _C), jnp.float32),
            pltpu.SemaphoreType.DMA,
        ],
    )
    def gk(cb_hbm, idx_hbm, out_hbm, idx_v, rows_v, sem):
        wid = lax.axis_index("s") * 2 + lax.axis_index("c")
        base = wid * _BPW
        pltpu.sync_copy(idx_hbm.at[wid], idx_v)
        for j in range(_JCH):
            pltpu.async_copy(cb_hbm.at[idx_v.at[j]], rows_v, sem).wait()
            pltpu.sync_copy(rows_v, out_hbm.at[pl.ds(base + j * 128, 128)])

    return gk(codebook, idx3)


def kernel(x, codebook):
    B, T, C = x.shape
    x_flat = x.reshape(-1, C)
    idx2, loss = _vq_tc(x_flat, codebook)
    idx_flat = idx2[:, 0]
    qe_flat = _sc_gather(codebook, idx_flat.reshape(_NW, _JCH, 128))
    qe = qe_flat.reshape(B, T, C)
    return (qe, loss[0, 0], idx_flat.reshape(B, T))
